# Initial kernel scaffold; baseline (speedup 1.0000x reference)
import jax
import jax.numpy as jnp
from jax.experimental import pallas as pl


def kernel(x, segment_ids, cos_flat, q_star, W_ih, W_hh, b_ih, b_hh):
    raise NotImplementedError("write your pallas kernel here")


# speedup vs baseline: 6.4491x; 6.4491x over previous
import jax
import jax.numpy as jnp
from jax.lax import Precision as _Prec
from jax.experimental import pallas as pl
from jax.experimental.pallas import tpu as pltpu

def _sigmoid(z):
    return 1.0 / (1.0 + jnp.exp(-z))


def _tanh(z):
    return 1.0 - 2.0 / (jnp.exp(2.0 * z) + 1.0)


IC = 200
STEPS = 3
NSEG = 16
NTOK = 32768
T = 2048
NC = NTOK // T


def _body(x_ref, segr_ref, cosr_ref, qstar_ref, wihT_ref, whhT_ref, bih_ref, bhh_ref,
          out_ref, e_ref):
    f32 = jnp.float32

    def seg_mask(c):
        seg = segr_ref[:, pl.ds(c * T, T)]
        return jax.lax.broadcasted_iota(jnp.int32, (NSEG, T), 0) == seg

    def x_chunk(c):
        return x_ref[pl.ds(c * T, T), :]

    def asit_step(c, acc):
        w = seg_mask(c).astype(f32) * cosr_ref[:, pl.ds(c * T, T)]
        return acc + jnp.dot(w, x_chunk(c),
                             preferred_element_type=f32,
                             precision=_Prec.HIGHEST)
    a_sit = jax.lax.fori_loop(0, NC, asit_step, jnp.zeros((NSEG, IC), f32))

    h = a_sit
    c_st = jnp.zeros((NSEG, IC), f32)
    q_star = qstar_ref[...]
    wihT = wihT_ref[...]
    whhT = whhT_ref[...]
    bih = bih_ref[...]
    bhh = bhh_ref[...]

    for _ in range(STEPS):
        gates = (jnp.dot(q_star, wihT, preferred_element_type=f32)
                 + bih
                 + jnp.dot(h, whhT, preferred_element_type=f32)
                 + bhh)
        i_g = _sigmoid(gates[:, 0 * IC:1 * IC])
        f_g = _sigmoid(gates[:, 1 * IC:2 * IC])
        g_g = _tanh(gates[:, 2 * IC:3 * IC])
        o_g = _sigmoid(gates[:, 3 * IC:4 * IC])
        c_st = f_g * c_st + i_g * g_g
        h = o_g * _tanh(c_st)
        q = h

        qT = jnp.swapaxes(q, 0, 1)

        def logits_step(c, m):
            ec = jnp.swapaxes(
                jnp.dot(x_chunk(c), qT, preferred_element_type=f32,
                        precision=_Prec.HIGHEST), 0, 1)
            e_ref[:, pl.ds(c * T, T)] = ec
            mc = jnp.max(jnp.where(seg_mask(c), ec, -jnp.inf), axis=1,
                         keepdims=True)
            return jnp.maximum(m, mc)
        m = jax.lax.fori_loop(0, NC, logits_step,
                              jnp.full((NSEG, 1), -jnp.inf, f32))
        m = jnp.where(jnp.isfinite(m), m, 0.0)

        def pool_step(c, carry):
            racc, d = carry
            ec = e_ref[:, pl.ds(c * T, T)]
            pc = jnp.exp(jnp.where(seg_mask(c), ec - m, -jnp.inf))
            d = d + jnp.sum(pc, axis=1, keepdims=True)
            racc = racc + jnp.dot(pc, x_chunk(c),
                                  preferred_element_type=f32,
                                  precision=_Prec.HIGHEST)
            return racc, d
        racc, d = jax.lax.fori_loop(
            0, NC, pool_step,
            (jnp.zeros((NSEG, IC), f32), jnp.zeros((NSEG, 1), f32)))
        r = racc / (d + 1e-16)
        q_star = jnp.concatenate([q, r], axis=1)

    out_ref[...] = q_star


def _run(x, segr, cosr, q_star, wihT, whhT, bih, bhh):
    return pl.pallas_call(
        _body,
        out_shape=jax.ShapeDtypeStruct((NSEG, 2 * IC), jnp.float32),
        scratch_shapes=[pltpu.VMEM((NSEG, NTOK), jnp.float32)],
    )(x, segr, cosr, q_star, wihT, whhT, bih, bhh)


def kernel(x, segment_ids, cos_flat, q_star, W_ih, W_hh, b_ih, b_hh):
    segr = segment_ids.astype(jnp.int32).reshape(1, NTOK)
    cosr = cos_flat.reshape(1, NTOK)
    wihT = W_ih.T
    whhT = W_hh.T
    bih = b_ih.reshape(1, 4 * IC)
    bhh = b_hh.reshape(1, 4 * IC)
    return _run(x, segr, cosr, q_star, wihT, whhT, bih, bhh)


# speedup vs baseline: 12.6399x; 1.9599x over previous
import jax
import jax.numpy as jnp
from jax.lax import Precision as _Prec
from jax.experimental import pallas as pl
from jax.experimental.pallas import tpu as pltpu

def _sigmoid(z):
    return 1.0 / (1.0 + jnp.exp(-z))


def _tanh(z):
    return 1.0 - 2.0 / (jnp.exp(2.0 * z) + 1.0)


IC = 200
STEPS = 3
NSEG = 16
NTOK = 32768
T = 2048
NC = NTOK // T


def _body(x_ref, segr_ref, cosr_ref, qstar_ref, wihT_ref, whhT_ref, bih_ref, bhh_ref,
          out_ref, e_ref):
    f32 = jnp.float32

    def seg_mask(c):
        seg = segr_ref[:, pl.ds(c * T, T)]
        return jax.lax.broadcasted_iota(jnp.int32, (NSEG, T), 0) == seg

    def x_chunk(c):
        return x_ref[pl.ds(c * T, T), :]

    def asit_step(c, acc):
        w = seg_mask(c).astype(f32) * cosr_ref[:, pl.ds(c * T, T)]
        return acc + jnp.dot(w, x_chunk(c),
                             preferred_element_type=f32,
                             precision=_Prec.HIGHEST)
    a_sit = jax.lax.fori_loop(0, NC, asit_step, jnp.zeros((NSEG, IC), f32))

    h = a_sit
    c_st = jnp.zeros((NSEG, IC), f32)
    q_star = qstar_ref[...]
    wihT = wihT_ref[...]
    whhT = whhT_ref[...]
    bih = bih_ref[...]
    bhh = bhh_ref[...]

    for _ in range(STEPS):
        gates = (jnp.dot(q_star, wihT, preferred_element_type=f32)
                 + bih
                 + jnp.dot(h, whhT, preferred_element_type=f32)
                 + bhh)
        i_g = _sigmoid(gates[:, 0 * IC:1 * IC])
        f_g = _sigmoid(gates[:, 1 * IC:2 * IC])
        g_g = _tanh(gates[:, 2 * IC:3 * IC])
        o_g = _sigmoid(gates[:, 3 * IC:4 * IC])
        c_st = f_g * c_st + i_g * g_g
        h = o_g * _tanh(c_st)
        q = h

        qT = jnp.swapaxes(q, 0, 1)

        def logits_step(c, m):
            ec = jnp.swapaxes(
                jnp.dot(x_chunk(c), qT, preferred_element_type=f32), 0, 1)
            e_ref[:, pl.ds(c * T, T)] = ec
            mc = jnp.max(jnp.where(seg_mask(c), ec, -jnp.inf), axis=1,
                         keepdims=True)
            return jnp.maximum(m, mc)
        m = jax.lax.fori_loop(0, NC, logits_step,
                              jnp.full((NSEG, 1), -jnp.inf, f32))
        m = jnp.where(jnp.isfinite(m), m, 0.0)

        def pool_step(c, carry):
            racc, d = carry
            ec = e_ref[:, pl.ds(c * T, T)]
            pc = jnp.exp(jnp.where(seg_mask(c), ec - m, -jnp.inf))
            d = d + jnp.sum(pc, axis=1, keepdims=True)
            racc = racc + jnp.dot(pc, x_chunk(c),
                                  preferred_element_type=f32)
            return racc, d
        racc, d = jax.lax.fori_loop(
            0, NC, pool_step,
            (jnp.zeros((NSEG, IC), f32), jnp.zeros((NSEG, 1), f32)))
        r = racc / (d + 1e-16)
        q_star = jnp.concatenate([q, r], axis=1)

    out_ref[...] = q_star


def _run(x, segr, cosr, q_star, wihT, whhT, bih, bhh):
    return pl.pallas_call(
        _body,
        out_shape=jax.ShapeDtypeStruct((NSEG, 2 * IC), jnp.float32),
        scratch_shapes=[pltpu.VMEM((NSEG, NTOK), jnp.float32)],
    )(x, segr, cosr, q_star, wihT, whhT, bih, bhh)


def kernel(x, segment_ids, cos_flat, q_star, W_ih, W_hh, b_ih, b_hh):
    segr = segment_ids.astype(jnp.int32).reshape(1, NTOK)
    cosr = cos_flat.reshape(1, NTOK)
    wihT = W_ih.T
    whhT = W_hh.T
    bih = b_ih.reshape(1, 4 * IC)
    bhh = b_hh.reshape(1, 4 * IC)
    return _run(x, segr, cosr, q_star, wihT, whhT, bih, bhh)
